# baseline (device time: 47840 ns/iter reference)
import jax
import jax.numpy as jnp
from jax import lax
from jax.experimental import pallas as pl
from jax.experimental.pallas import tpu as pltpu

N_DEV = 4
SQ = 512
D = 1024
DH = 128
HQ_LOCAL = 8
GROUP = 4
KV_LOCAL = HQ_LOCAL // GROUP
SCALE = 0.08838834764831843
NBLK = 2
BLK = SQ // NBLK


def _body(x_ref, wq_ref, wo_ref, wk_ref, wv_ref, out_ref,
          ybuf, pl_buf, pr_buf, pd_buf,
          sa_send, sa_recv, sb_send, sb_recv, sc_send, sc_recv):
    my_pos = lax.axis_index("i")
    left = (my_pos - 1) % N_DEV
    right = (my_pos + 1) % N_DEV
    diag = (my_pos + 2) % N_DEV

    barrier_sem = pltpu.get_barrier_semaphore()
    for nbr in [left, right, diag]:
        pl.semaphore_signal(
            barrier_sem, inc=1,
            device_id=(nbr,), device_id_type=pl.DeviceIdType.MESH,
        )
    pl.semaphore_wait(barrier_sem, 3)

    xb = x_ref[...].astype(jnp.bfloat16)
    wq = wq_ref[...].astype(jnp.bfloat16)
    wo = wo_ref[...].astype(jnp.bfloat16)
    kv_col = my_pos * (KV_LOCAL * DH)
    wk = wk_ref[:, pl.ds(kv_col, KV_LOCAL * DH)].astype(jnp.bfloat16)
    wv = wv_ref[:, pl.ds(kv_col, KV_LOCAL * DH)].astype(jnp.bfloat16)

    q = jnp.dot(xb, wq, preferred_element_type=jnp.float32).astype(jnp.bfloat16)
    k = jnp.dot(xb, wk, preferred_element_type=jnp.float32).astype(jnp.bfloat16)
    v = jnp.dot(xb, wv, preferred_element_type=jnp.float32).astype(jnp.bfloat16)

    def make_rdma(b, dst_buf, send_sems, recv_sems, dev):
        return pltpu.make_async_remote_copy(
            src_ref=ybuf.at[b],
            dst_ref=dst_buf.at[b],
            send_sem=send_sems.at[b],
            recv_sem=recv_sems.at[b],
            device_id=(dev,),
            device_id_type=pl.DeviceIdType.MESH,
        )

    rdmas = []
    for b in range(NBLK):
        r0 = b * BLK
        for h in range(HQ_LOCAL):
            qh = q[r0:r0 + BLK, h * DH:(h + 1) * DH]
            kvh = h // GROUP
            kh = k[:, kvh * DH:(kvh + 1) * DH]
            vh = v[:, kvh * DH:(kvh + 1) * DH]
            s = lax.dot_general(
                qh, kh, (((1,), (1,)), ((), ())),
                preferred_element_type=jnp.float32,
            ) * SCALE
            m = jnp.max(s, axis=1, keepdims=True)
            p = jnp.exp(s - m)
            l = jnp.sum(p, axis=1, keepdims=True)
            o = jnp.dot(p.astype(jnp.bfloat16), vh,
                        preferred_element_type=jnp.float32)
            if h == 0:
                attn = (o / l).astype(jnp.bfloat16)
            else:
                attn = jnp.concatenate(
                    [attn, (o / l).astype(jnp.bfloat16)], axis=1)

        yb = jnp.dot(attn, wo, preferred_element_type=jnp.float32)
        out_ref[r0:r0 + BLK, :] = yb
        ybuf[b] = yb.astype(jnp.bfloat16)

        ra = make_rdma(b, pl_buf, sa_send, sa_recv, right)
        rb = make_rdma(b, pr_buf, sb_send, sb_recv, left)
        rc = make_rdma(b, pd_buf, sc_send, sc_recv, diag)
        ra.start()
        rb.start()
        rc.start()
        rdmas.append((ra, rb, rc))

    for b in range(NBLK):
        r0 = b * BLK
        ra, rb, rc = rdmas[b]
        ra.wait()
        rb.wait()
        rc.wait()
        out_ref[r0:r0 + BLK, :] += (
            pl_buf[b].astype(jnp.float32)
            + pr_buf[b].astype(jnp.float32)
            + pd_buf[b].astype(jnp.float32)
        )


def kernel(x, Wq, Wo, Wk, Wv):
    x2 = x.reshape(SQ, D)
    out = pl.pallas_call(
        _body,
        out_shape=jax.ShapeDtypeStruct((SQ, D), jnp.float32),
        in_specs=[pl.BlockSpec(memory_space=pltpu.VMEM)] * 5,
        out_specs=pl.BlockSpec(memory_space=pltpu.VMEM),
        scratch_shapes=[
            pltpu.VMEM((NBLK, BLK, D), jnp.bfloat16),
            pltpu.VMEM((NBLK, BLK, D), jnp.bfloat16),
            pltpu.VMEM((NBLK, BLK, D), jnp.bfloat16),
            pltpu.VMEM((NBLK, BLK, D), jnp.bfloat16),
            pltpu.SemaphoreType.DMA((NBLK,)),
            pltpu.SemaphoreType.DMA((NBLK,)),
            pltpu.SemaphoreType.DMA((NBLK,)),
            pltpu.SemaphoreType.DMA((NBLK,)),
            pltpu.SemaphoreType.DMA((NBLK,)),
            pltpu.SemaphoreType.DMA((NBLK,)),
        ],
        compiler_params=pltpu.CompilerParams(collective_id=0),
    )(x2, Wq, Wo, Wk, Wv)
    return out.reshape(1, SQ, D)


# device time: 39106 ns/iter; 1.2233x vs baseline; 1.2233x over previous
import jax
import jax.numpy as jnp
from jax import lax
from jax.experimental import pallas as pl
from jax.experimental.pallas import tpu as pltpu

N_DEV = 4
SQ = 512
D = 1024
DH = 128
HQ_LOCAL = 8
GROUP = 4
KV_LOCAL = HQ_LOCAL // GROUP
SCALE = 0.08838834764831843
NBLK = 4
BLK = SQ // NBLK
COMM = True


def _body(x_ref, wq_ref, wo_ref, wk_ref, wv_ref, out_ref,
          attn_ref, ybuf, psum_buf, recv1, recv2,
          s1_send, s1_recv, s2_send, s2_recv):
    my_pos = lax.axis_index("i")
    left = (my_pos - 1) % N_DEV
    right = (my_pos + 1) % N_DEV
    is_even = (my_pos % 2) == 0
    dev1 = jnp.where(is_even, right, left)
    dev2 = jnp.where(is_even, left, right)

    barrier_sem = pltpu.get_barrier_semaphore()
    for nbr in [left, right]:
        pl.semaphore_signal(
            barrier_sem, inc=1,
            device_id=(nbr,), device_id_type=pl.DeviceIdType.MESH,
        )
    pl.semaphore_wait(barrier_sem, 2)

    xb = x_ref[...].astype(jnp.bfloat16)
    wq = (wq_ref[...] * SCALE).astype(jnp.bfloat16)
    wo = wo_ref[...].astype(jnp.bfloat16)
    kv_col = my_pos * (KV_LOCAL * DH)
    wk = wk_ref[:, pl.ds(kv_col, KV_LOCAL * DH)].astype(jnp.bfloat16)
    wv = wv_ref[:, pl.ds(kv_col, KV_LOCAL * DH)].astype(jnp.bfloat16)

    q = jnp.dot(xb, wq, preferred_element_type=jnp.float32).astype(jnp.bfloat16)
    k = jnp.dot(xb, wk, preferred_element_type=jnp.float32).astype(jnp.bfloat16)
    v = jnp.dot(xb, wv, preferred_element_type=jnp.float32).astype(jnp.bfloat16)

    def exchange(src_buf, dst_buf, send_sems, recv_sems, b, dev):
        r = pltpu.make_async_remote_copy(
            src_ref=src_buf.at[b],
            dst_ref=dst_buf.at[b],
            send_sem=send_sems.at[b],
            recv_sem=recv_sems.at[b],
            device_id=(dev,),
            device_id_type=pl.DeviceIdType.MESH,
        )
        r.start()
        return r

    ph1 = [None] * NBLK
    ph2 = [None] * NBLK

    def advance(j):
        r0 = j * BLK
        ph1[j].wait()
        ps = out_ref[r0:r0 + BLK, :] + recv1[j].astype(jnp.float32)
        out_ref[r0:r0 + BLK, :] = ps
        psum_buf[j] = ps.astype(jnp.bfloat16)
        ph2[j] = exchange(psum_buf, recv2, s2_send, s2_recv, j, dev2)

    def finish(j):
        r0 = j * BLK
        ph2[j].wait()
        out_ref[r0:r0 + BLK, :] += recv2[j].astype(jnp.float32)

    for b in range(NBLK):
        r0 = b * BLK
        for h in range(HQ_LOCAL):
            qh = q[r0:r0 + BLK, h * DH:(h + 1) * DH]
            kvh = h // GROUP
            kh = k[:, kvh * DH:(kvh + 1) * DH]
            vh = v[:, kvh * DH:(kvh + 1) * DH]
            s = lax.dot_general(
                qh, kh, (((1,), (1,)), ((), ())),
                preferred_element_type=jnp.float32,
            )
            p = jnp.exp(s)
            rl = 1.0 / jnp.sum(p, axis=1, keepdims=True)
            o = jnp.dot(p.astype(jnp.bfloat16), vh,
                        preferred_element_type=jnp.float32)
            attn_ref[:, h * DH:(h + 1) * DH] = (o * rl).astype(jnp.bfloat16)

        yb = jnp.dot(attn_ref[...], wo, preferred_element_type=jnp.float32)
        out_ref[r0:r0 + BLK, :] = yb
        if COMM:
            ybuf[b] = yb.astype(jnp.bfloat16)
            ph1[b] = exchange(ybuf, recv1, s1_send, s1_recv, b, dev1)
            if b >= 1:
                advance(b - 1)
            if b >= 2:
                finish(b - 2)

    if COMM:
        advance(NBLK - 1)
        finish(NBLK - 2)
        finish(NBLK - 1)


def kernel(x, Wq, Wo, Wk, Wv):
    x2 = x.reshape(SQ, D)
    out = pl.pallas_call(
        _body,
        out_shape=jax.ShapeDtypeStruct((SQ, D), jnp.float32),
        in_specs=[pl.BlockSpec(memory_space=pltpu.VMEM)] * 5,
        out_specs=pl.BlockSpec(memory_space=pltpu.VMEM),
        scratch_shapes=[
            pltpu.VMEM((BLK, D), jnp.bfloat16),
            pltpu.VMEM((NBLK, BLK, D), jnp.bfloat16),
            pltpu.VMEM((NBLK, BLK, D), jnp.bfloat16),
            pltpu.VMEM((NBLK, BLK, D), jnp.bfloat16),
            pltpu.VMEM((NBLK, BLK, D), jnp.bfloat16),
            pltpu.SemaphoreType.DMA((NBLK,)),
            pltpu.SemaphoreType.DMA((NBLK,)),
            pltpu.SemaphoreType.DMA((NBLK,)),
            pltpu.SemaphoreType.DMA((NBLK,)),
        ],
        compiler_params=pltpu.CompilerParams(collective_id=0),
    )(x2, Wq, Wo, Wk, Wv)
    return out.reshape(1, SQ, D)


# device time: 35970 ns/iter; 1.3300x vs baseline; 1.0872x over previous
import jax
import jax.numpy as jnp
from jax import lax
from jax.experimental import pallas as pl
from jax.experimental.pallas import tpu as pltpu

N_DEV = 4
SQ = 512
D = 1024
DH = 128
HQ_LOCAL = 8
GROUP = 4
KV_LOCAL = HQ_LOCAL // GROUP
KV_D = KV_LOCAL * DH
SCALE = 0.08838834764831843
LOG2E = 1.4426950408889634
ROWS = (128, 128, 128, 128)
START = (0, 128, 256, 384)
NBLK = len(ROWS)
BLK_MAX = max(ROWS)
COMM = True


def _body(x_ref, wqkv_ref, wo_ref, out_ref,
          ybuf, psum_buf, recv1, recv2,
          s1_send, s1_recv, s2_send, s2_recv):
    my_pos = lax.axis_index("i")
    left = (my_pos - 1) % N_DEV
    right = (my_pos + 1) % N_DEV
    is_even = (my_pos % 2) == 0
    dev1 = jnp.where(is_even, right, left)
    dev2 = jnp.where(is_even, left, right)

    barrier_sem = pltpu.get_barrier_semaphore()
    for nbr in [left, right]:
        pl.semaphore_signal(
            barrier_sem, inc=1,
            device_id=(nbr,), device_id_type=pl.DeviceIdType.MESH,
        )
    pl.semaphore_wait(barrier_sem, 2)

    qkv = jnp.dot(x_ref[...], wqkv_ref[...],
                  preferred_element_type=jnp.float32).astype(jnp.bfloat16)
    q = qkv[:, :D]
    k = qkv[:, D:D + KV_D]
    v = qkv[:, D + KV_D:]
    wo = wo_ref[...]
    ones = jnp.ones((SQ, DH), jnp.bfloat16)
    vext = [jnp.concatenate([v[:, g * DH:(g + 1) * DH], ones], axis=1)
            for g in range(KV_LOCAL)]

    def exchange(src_buf, dst_buf, send_sems, recv_sems, b, dev):
        nrows = ROWS[b]
        r = pltpu.make_async_remote_copy(
            src_ref=src_buf.at[b, pl.ds(0, nrows)],
            dst_ref=dst_buf.at[b, pl.ds(0, nrows)],
            send_sem=send_sems.at[b],
            recv_sem=recv_sems.at[b],
            device_id=(dev,),
            device_id_type=pl.DeviceIdType.MESH,
        )
        r.start()
        return r

    ph1 = [None] * NBLK
    ph2 = [None] * NBLK

    def advance(j):
        nr = ROWS[j]
        ph1[j].wait()
        psum_buf[j, :nr] = ybuf[j, :nr] + recv1[j, :nr]
        ph2[j] = exchange(psum_buf, recv2, s2_send, s2_recv, j, dev2)

    def finish(j):
        r0, nr = START[j], ROWS[j]
        ph2[j].wait()
        out_ref[r0:r0 + nr, :] = psum_buf[j, :nr] + recv2[j, :nr]

    for b in range(NBLK):
        r0, BLK = START[b], ROWS[b]
        attn_halves = []
        for g in range(KV_LOCAL):
            qg = q[r0:r0 + BLK, g * GROUP * DH:(g + 1) * GROUP * DH]
            q4 = qg.reshape(GROUP * BLK, DH)
            kh = k[:, g * DH:(g + 1) * DH]
            vh = vext[g]
            s = lax.dot_general(
                q4, kh, (((1,), (1,)), ((), ())),
                preferred_element_type=jnp.float32,
            )
            p = jnp.exp2(s.astype(jnp.bfloat16))
            o1 = jnp.dot(p, vh,
                         preferred_element_type=jnp.float32)
            rl = 1.0 / o1[:, DH:DH + 1]
            attn_halves.append(
                (o1[:, :DH] * rl).astype(jnp.bfloat16).reshape(BLK, GROUP * DH))

        attn_b = jnp.concatenate(attn_halves, axis=1)
        yb = jnp.dot(attn_b, wo, preferred_element_type=jnp.float32)

        if COMM:
            ybuf[b, :BLK] = yb.astype(jnp.bfloat16)
            ph1[b] = exchange(ybuf, recv1, s1_send, s1_recv, b, dev1)
            if b >= 1:
                advance(b - 1)
            if b >= 2:
                finish(b - 2)
        else:
            out_ref[r0:r0 + BLK, :] = yb.astype(jnp.bfloat16)

    if COMM:
        advance(NBLK - 1)
        finish(NBLK - 2)
        finish(NBLK - 1)


def kernel(x, Wq, Wo, Wk, Wv):
    my_pos = lax.axis_index("i")
    kv_col = my_pos * KV_D
    wk = lax.dynamic_slice(Wk, (0, kv_col), (D, KV_D))
    wv = lax.dynamic_slice(Wv, (0, kv_col), (D, KV_D))
    wqkv = jnp.concatenate(
        [Wq * (SCALE * LOG2E), wk, wv], axis=1).astype(jnp.bfloat16)
    wo = Wo.astype(jnp.bfloat16)
    xb = x.reshape(SQ, D).astype(jnp.bfloat16)

    out = pl.pallas_call(
        _body,
        out_shape=jax.ShapeDtypeStruct((SQ, D), jnp.bfloat16),
        in_specs=[pl.BlockSpec(memory_space=pltpu.VMEM)] * 3,
        out_specs=pl.BlockSpec(memory_space=pltpu.VMEM),
        scratch_shapes=[
            pltpu.VMEM((NBLK, BLK_MAX, D), jnp.bfloat16),
            pltpu.VMEM((NBLK, BLK_MAX, D), jnp.bfloat16),
            pltpu.VMEM((NBLK, BLK_MAX, D), jnp.bfloat16),
            pltpu.VMEM((NBLK, BLK_MAX, D), jnp.bfloat16),
            pltpu.SemaphoreType.DMA((NBLK,)),
            pltpu.SemaphoreType.DMA((NBLK,)),
            pltpu.SemaphoreType.DMA((NBLK,)),
            pltpu.SemaphoreType.DMA((NBLK,)),
        ],
        compiler_params=pltpu.CompilerParams(collective_id=0),
    )(xb, wqkv, wo)
    return out.reshape(1, SQ, D)
